# baseline (speedup 1.0000x reference)
import functools

import jax
import jax.numpy as jnp
from jax import lax
from jax.experimental import pallas as pl
from jax.experimental.pallas import tpu as pltpu
from jax.experimental.pallas import tpu_sc as plsc

_LANES = 16
_IDXW = 100


@functools.lru_cache(maxsize=None)
def _build_embed(rows, emb, seq):
    info = plsc.get_sparse_core_info()
    nc, ns = info.num_cores, info.num_subcores
    nw = nc * ns
    assert rows % nw == 0
    rpw = rows // nw
    chunk = 4 * seq
    assert rpw % chunk == 0 and chunk % _IDXW == 0
    nch = rpw // chunk
    ng = chunk // _IDXW
    nvec = emb // _LANES
    assert emb % _LANES == 0
    nrep = chunk // seq

    mesh = plsc.VectorSubcoreMesh(core_axis_name="c", subcore_axis_name="s")

    @functools.partial(
        pl.kernel,
        mesh=mesh,
        compiler_params=pltpu.CompilerParams(use_tc_tiling_on_sc=False),
        out_type=jax.ShapeDtypeStruct((rows, emb), jnp.float32),
        scratch_types=[
            pltpu.VMEM((2 * ng, _IDXW), jnp.int32),
            pltpu.VMEM((2 * chunk, emb), jnp.float32),
            pltpu.VMEM((seq, emb), jnp.float32),
            pltpu.SemaphoreType.DMA,
            pltpu.SemaphoreType.DMA,
        ],
    )
    def k(idx_hbm, table_hbm, pos_hbm, out_hbm, idx_v, rows_v, pos_v, sem0, sem1):
        wid = lax.axis_index("s") * nc + lax.axis_index("c")
        base = wid * rpw
        sems = (sem0, sem1)
        pltpu.sync_copy(pos_hbm.at[pl.ds(0, seq)], pos_v)

        def start_chunk(c, p):
            irow = pl.multiple_of(base // _IDXW + c * ng, 8)
            pltpu.sync_copy(
                idx_hbm.at[pl.ds(irow, ng)], idx_v.at[pl.ds(p * ng, ng)]
            )
            return [
                pltpu.async_copy(
                    table_hbm.at[idx_v.at[p * ng + g]],
                    rows_v.at[pl.ds(p * chunk + g * _IDXW, _IDXW)],
                    sems[p],
                )
                for g in range(ng)
            ]

        pending = start_chunk(0, 0)
        for c in range(nch):
            p = c % 2
            for cp in pending:
                cp.wait()
            if c + 1 < nch:
                pending = start_chunk(c + 1, 1 - p)

            def add_body(s, carry):
                for e in range(nvec):
                    pv = pos_v[s, pl.ds(e * _LANES, _LANES)]
                    for q in range(nrep):
                        r = p * chunk + q * seq + s
                        rows_v[r, pl.ds(e * _LANES, _LANES)] = (
                            rows_v[r, pl.ds(e * _LANES, _LANES)] + pv
                        )
                return carry

            lax.fori_loop(0, seq, add_body, None)
            r0 = pl.multiple_of(base + c * chunk, 8)
            pltpu.sync_copy(
                rows_v.at[pl.ds(p * chunk, chunk)], out_hbm.at[pl.ds(r0, chunk)]
            )

    return k


_PACK_BT = 128


@functools.lru_cache(maxsize=None)
def _build_pack(vocab, emb):
    grid = (vocab + _PACK_BT - 1) // _PACK_BT

    def body(x_ref, se_ref, so_ref, o_ref):
        y = x_ref[...].T
        e = jnp.dot(se_ref[...], y, preferred_element_type=jnp.float32)
        o = jnp.dot(so_ref[...], y, preferred_element_type=jnp.float32)
        o_ref[...] = jnp.concatenate([e, o], axis=1)

    return pl.pallas_call(
        body,
        grid=(grid,),
        in_specs=[
            pl.BlockSpec((emb, _PACK_BT), lambda i: (0, i)),
            pl.BlockSpec((_PACK_BT // 2, _PACK_BT), lambda i: (0, 0)),
            pl.BlockSpec((_PACK_BT // 2, _PACK_BT), lambda i: (0, 0)),
        ],
        out_specs=pl.BlockSpec((_PACK_BT // 2, 2 * emb), lambda i: (i, 0)),
        out_shape=jax.ShapeDtypeStruct((vocab // 2, 2 * emb), jnp.float32),
    )


def kernel(input_tokens, token_table, pos_table):
    b, s = input_tokens.shape
    vocab, emb = token_table.shape
    rows = b * s
    idx = input_tokens.astype(jnp.int32).reshape(rows // _IDXW, _IDXW)
    half = _PACK_BT // 2
    r_ids = lax.broadcasted_iota(jnp.int32, (half, _PACK_BT), 0)
    t_ids = lax.broadcasted_iota(jnp.int32, (half, _PACK_BT), 1)
    sel_e = (t_ids == 2 * r_ids).astype(jnp.float32)
    sel_o = (t_ids == 2 * r_ids + 1).astype(jnp.float32)
    packed = _build_pack(vocab, emb)(token_table.T, sel_e, sel_o)
    tbl_lin = packed.reshape(vocab, emb)
    out = _build_embed(rows, emb, s)(idx, tbl_lin, pos_table)
    return out.reshape(b, s, emb)


# speedup vs baseline: 9.1450x; 9.1450x over previous
import functools

import jax
import jax.numpy as jnp
from jax import lax
from jax.experimental import pallas as pl
from jax.experimental.pallas import tpu as pltpu
from jax.experimental.pallas import tpu_sc as plsc

_LANES = 16
_IDXW = 100


@functools.lru_cache(maxsize=None)
def _build_embed(rows, emb, seq):
    info = plsc.get_sparse_core_info()
    nc, ns = info.num_cores, info.num_subcores
    nw = nc * ns
    assert rows % nw == 0
    rpw = rows // nw
    chunk = 4 * seq
    assert rpw % chunk == 0 and chunk % _IDXW == 0
    nch = rpw // chunk
    ng = chunk // _IDXW
    nvec = emb // _LANES
    assert emb % _LANES == 0
    nrep = chunk // seq

    mesh = plsc.VectorSubcoreMesh(core_axis_name="c", subcore_axis_name="s")

    @functools.partial(
        pl.kernel,
        mesh=mesh,
        compiler_params=pltpu.CompilerParams(use_tc_tiling_on_sc=False),
        out_type=jax.ShapeDtypeStruct((rows, emb), jnp.float32),
        scratch_types=[
            pltpu.VMEM((2 * ng, _IDXW), jnp.int32),
            pltpu.VMEM((2 * chunk, emb), jnp.float32),
            pltpu.VMEM((seq, emb), jnp.float32),
            pltpu.SemaphoreType.DMA,
            pltpu.SemaphoreType.DMA,
        ],
    )
    def k(idx_hbm, table_hbm, pos_hbm, out_hbm, idx_v, rows_v, pos_v, sem0, sem1):
        wid = lax.axis_index("s") * nc + lax.axis_index("c")
        base = wid * rpw
        sems = (sem0, sem1)
        pltpu.sync_copy(pos_hbm.at[pl.ds(0, seq)], pos_v)

        def start_chunk(c, p):
            irow = pl.multiple_of(base // _IDXW + c * ng, 8)
            pltpu.sync_copy(
                idx_hbm.at[pl.ds(irow, ng)], idx_v.at[pl.ds(p * ng, ng)]
            )
            return [
                pltpu.async_copy(
                    table_hbm.at[idx_v.at[p * ng + g]],
                    rows_v.at[pl.ds(p * chunk + g * _IDXW, _IDXW)],
                    sems[p],
                )
                for g in range(ng)
            ]

        pending = start_chunk(0, 0)
        for c in range(nch):
            p = c % 2
            for cp in pending:
                cp.wait()
            if c + 1 < nch:
                pending = start_chunk(c + 1, 1 - p)

            def add_body(s, carry):
                for e in range(nvec):
                    pv = pos_v[s, pl.ds(e * _LANES, _LANES)]
                    for q in range(nrep):
                        r = p * chunk + q * seq + s
                        rows_v[r, pl.ds(e * _LANES, _LANES)] = (
                            rows_v[r, pl.ds(e * _LANES, _LANES)] + pv
                        )
                return carry

            lax.fori_loop(0, seq, add_body, None)
            r0 = pl.multiple_of(base + c * chunk, 8)
            pltpu.sync_copy(
                rows_v.at[pl.ds(p * chunk, chunk)], out_hbm.at[pl.ds(r0, chunk)]
            )

    return k


_PACK_BT = 128
_PACK_REP = 32


@functools.lru_cache(maxsize=None)
def _build_pack(vocab, emb):
    bt = _PACK_BT * _PACK_REP
    grid = (vocab + bt - 1) // bt
    half = _PACK_BT // 2

    def body(x_ref, se_ref, so_ref, o_ref):
        se = se_ref[...]
        so = so_ref[...]
        for j in range(_PACK_REP):
            y = x_ref[:, j * _PACK_BT:(j + 1) * _PACK_BT].T
            o_ref[j * half:(j + 1) * half, 0:emb] = jnp.dot(
                se, y, preferred_element_type=jnp.float32
            )
            o_ref[j * half:(j + 1) * half, emb:2 * emb] = jnp.dot(
                so, y, preferred_element_type=jnp.float32
            )

    return pl.pallas_call(
        body,
        grid=(grid,),
        in_specs=[
            pl.BlockSpec((emb, bt), lambda i: (0, i)),
            pl.BlockSpec((half, _PACK_BT), lambda i: (0, 0)),
            pl.BlockSpec((half, _PACK_BT), lambda i: (0, 0)),
        ],
        out_specs=pl.BlockSpec((bt // 2, 2 * emb), lambda i: (i, 0)),
        out_shape=jax.ShapeDtypeStruct((vocab // 2, 2 * emb), jnp.float32),
    )


def kernel(input_tokens, token_table, pos_table):
    b, s = input_tokens.shape
    vocab, emb = token_table.shape
    rows = b * s
    idx = input_tokens.astype(jnp.int32).reshape(rows // _IDXW, _IDXW)
    half = _PACK_BT // 2
    r_ids = lax.broadcasted_iota(jnp.int32, (half, _PACK_BT), 0)
    t_ids = lax.broadcasted_iota(jnp.int32, (half, _PACK_BT), 1)
    sel_e = (t_ids == 2 * r_ids).astype(jnp.float32)
    sel_o = (t_ids == 2 * r_ids + 1).astype(jnp.float32)
    packed = _build_pack(vocab, emb)(token_table.T, sel_e, sel_o)
    tbl_lin = packed.reshape(vocab, emb)
    out = _build_embed(rows, emb, s)(idx, tbl_lin, pos_table)
    return out.reshape(b, s, emb)


# speedup vs baseline: 10.7354x; 1.1739x over previous
import functools

import jax
import jax.numpy as jnp
from jax import lax
from jax.experimental import pallas as pl
from jax.experimental.pallas import tpu as pltpu
from jax.experimental.pallas import tpu_sc as plsc

_LANES = 16
_IDXW = 100


@functools.lru_cache(maxsize=None)
def _build_embed(rows, emb, seq):
    info = plsc.get_sparse_core_info()
    nc, ns = info.num_cores, info.num_subcores
    nw = nc * ns
    assert rows % nw == 0
    rpw = rows // nw
    chunk = 4 * seq
    assert rpw % chunk == 0 and chunk % _IDXW == 0
    nch = rpw // chunk
    ng = chunk // _IDXW
    nvec = emb // _LANES
    assert emb % _LANES == 0
    nrep = chunk // seq

    mesh = plsc.VectorSubcoreMesh(core_axis_name="c", subcore_axis_name="s")

    @functools.partial(
        pl.kernel,
        mesh=mesh,
        compiler_params=pltpu.CompilerParams(use_tc_tiling_on_sc=False),
        out_type=jax.ShapeDtypeStruct((rows, emb), jnp.float32),
        scratch_types=[
            pltpu.VMEM((2 * ng, _IDXW), jnp.int32),
            pltpu.VMEM((2 * chunk, emb), jnp.float32),
            pltpu.VMEM((seq, emb), jnp.float32),
            pltpu.SemaphoreType.DMA,
            pltpu.SemaphoreType.DMA,
        ],
    )
    def k(idx_hbm, table_hbm, pos_hbm, out_hbm, idx_v, rows_v, pos_v, sem0, sem1):
        wid = lax.axis_index("s") * nc + lax.axis_index("c")
        base = wid * rpw
        sems = (sem0, sem1)
        pltpu.sync_copy(pos_hbm.at[pl.ds(0, seq)], pos_v)

        def start_chunk(c, p):
            irow = pl.multiple_of(base // _IDXW + c * ng, 8)
            pltpu.sync_copy(
                idx_hbm.at[pl.ds(irow, ng)], idx_v.at[pl.ds(p * ng, ng)]
            )
            return [
                pltpu.async_copy(
                    table_hbm.at[idx_v.at[p * ng + g]],
                    rows_v.at[pl.ds(p * chunk + g * _IDXW, _IDXW)],
                    sems[p],
                )
                for g in range(ng)
            ]

        pending = start_chunk(0, 0)
        for c in range(nch):
            p = c % 2
            for cp in pending:
                cp.wait()
            if c + 1 < nch:
                pending = start_chunk(c + 1, 1 - p)

            def add_body(s, carry):
                for e in range(nvec):
                    pv = pos_v[s, pl.ds(e * _LANES, _LANES)]
                    for q in range(nrep):
                        r = p * chunk + q * seq + s
                        rows_v[r, pl.ds(e * _LANES, _LANES)] = (
                            rows_v[r, pl.ds(e * _LANES, _LANES)] + pv
                        )
                return carry

            lax.fori_loop(0, seq, add_body, None)
            r0 = pl.multiple_of(base + c * chunk, 8)
            pltpu.sync_copy(
                rows_v.at[pl.ds(p * chunk, chunk)], out_hbm.at[pl.ds(r0, chunk)]
            )

    return k


_PACK_BT = 128
_PACK_REP = 64


@functools.lru_cache(maxsize=None)
def _build_pack(vocab, emb):
    bt = _PACK_BT * _PACK_REP
    grid = (vocab + bt - 1) // bt
    half = _PACK_BT // 2

    def body(x_ref, se_ref, so_ref, o_ref):
        se = se_ref[...]
        so = so_ref[...]
        dn = (((1,), (1,)), ((), ()))
        for j in range(_PACK_REP):
            x = x_ref[:, j * _PACK_BT:(j + 1) * _PACK_BT]
            o_ref[j * half:(j + 1) * half, 0:emb] = lax.dot_general(
                se, x, dn, preferred_element_type=jnp.float32
            )
            o_ref[j * half:(j + 1) * half, emb:2 * emb] = lax.dot_general(
                so, x, dn, preferred_element_type=jnp.float32
            )

    return pl.pallas_call(
        body,
        grid=(grid,),
        in_specs=[
            pl.BlockSpec((emb, bt), lambda i: (0, i)),
            pl.BlockSpec((half, _PACK_BT), lambda i: (0, 0)),
            pl.BlockSpec((half, _PACK_BT), lambda i: (0, 0)),
        ],
        out_specs=pl.BlockSpec((bt // 2, 2 * emb), lambda i: (i, 0)),
        out_shape=jax.ShapeDtypeStruct((vocab // 2, 2 * emb), jnp.float32),
    )


def kernel(input_tokens, token_table, pos_table):
    b, s = input_tokens.shape
    vocab, emb = token_table.shape
    rows = b * s
    idx = input_tokens.astype(jnp.int32).reshape(rows // _IDXW, _IDXW)
    half = _PACK_BT // 2
    r_ids = lax.broadcasted_iota(jnp.int32, (half, _PACK_BT), 0)
    t_ids = lax.broadcasted_iota(jnp.int32, (half, _PACK_BT), 1)
    sel_e = (t_ids == 2 * r_ids).astype(jnp.float32)
    sel_o = (t_ids == 2 * r_ids + 1).astype(jnp.float32)
    packed = _build_pack(vocab, emb)(token_table.T, sel_e, sel_o)
    tbl_lin = packed.reshape(vocab, emb)
    out = _build_embed(rows, emb, s)(idx, tbl_lin, pos_table)
    return out.reshape(b, s, emb)


# speedup vs baseline: 14.2818x; 1.3303x over previous
import functools

import jax
import jax.numpy as jnp
from jax import lax
from jax.experimental import pallas as pl
from jax.experimental.pallas import tpu as pltpu
from jax.experimental.pallas import tpu_sc as plsc

_LANES = 16
_IDXW = 100


@functools.lru_cache(maxsize=None)
def _build_embed(rows, emb, seq):
    info = plsc.get_sparse_core_info()
    nc, ns = info.num_cores, info.num_subcores
    nw = nc * ns
    assert rows % nw == 0
    rpw = rows // nw
    chunk = 4 * seq
    assert rpw % chunk == 0 and chunk % _IDXW == 0
    nch = rpw // chunk
    ng = chunk // _IDXW
    nvec = emb // _LANES
    assert emb % _LANES == 0
    nrep = chunk // seq

    mesh = plsc.VectorSubcoreMesh(core_axis_name="c", subcore_axis_name="s")

    @functools.partial(
        pl.kernel,
        mesh=mesh,
        compiler_params=pltpu.CompilerParams(use_tc_tiling_on_sc=False),
        out_type=jax.ShapeDtypeStruct((rows, 2 * emb), jnp.float32),
        scratch_types=[
            pltpu.VMEM((2 * ng, _IDXW), jnp.int32),
            pltpu.VMEM((2 * chunk, emb), jnp.float32),
            pltpu.VMEM((seq, emb), jnp.float32),
            pltpu.SemaphoreType.DMA,
            pltpu.SemaphoreType.DMA,
        ],
    )
    def k(idx_hbm, table_hbm, pos_hbm, out_hbm, idx_v, rows_v, pos_v, sem0, sem1):
        wid = lax.axis_index("s") * nc + lax.axis_index("c")
        base = wid * rpw
        sems = (sem0, sem1)
        pltpu.sync_copy(pos_hbm.at[pl.ds(0, seq)], pos_v)

        def start_chunk(c, p):
            irow = pl.multiple_of(base // _IDXW + c * ng, 8)
            pltpu.sync_copy(
                idx_hbm.at[pl.ds(irow, ng)], idx_v.at[pl.ds(p * ng, ng)]
            )
            return [
                pltpu.async_copy(
                    table_hbm.at[idx_v.at[p * ng + g]],
                    rows_v.at[pl.ds(p * chunk + g * _IDXW, _IDXW)],
                    sems[p],
                )
                for g in range(ng)
            ]

        pending = start_chunk(0, 0)
        for c in range(nch):
            p = c % 2
            for cp in pending:
                cp.wait()
            if c + 1 < nch:
                pending = start_chunk(c + 1, 1 - p)

            def add_body(s, carry):
                for e in range(nvec):
                    pv = pos_v[s, pl.ds(e * _LANES, _LANES)]
                    for q in range(nrep):
                        r = p * chunk + q * seq + s
                        rows_v[r, pl.ds(e * _LANES, _LANES)] = (
                            rows_v[r, pl.ds(e * _LANES, _LANES)] + pv
                        )
                return carry

            lax.fori_loop(0, seq, add_body, None)
            r0 = pl.multiple_of(base + c * chunk, 8)
            pltpu.sync_copy(
                rows_v.at[pl.ds(p * chunk, chunk)],
                out_hbm.at[pl.ds(r0, chunk), pl.ds(0, emb)],
            )

    return k


_PACK_BT = 128
_PACK_REP = 64


@functools.lru_cache(maxsize=None)
def _build_pack(vocab, emb):
    bt = _PACK_BT * _PACK_REP
    grid = (vocab + bt - 1) // bt
    half = _PACK_BT // 2

    def body(x_ref, s_ref, o_ref):
        sel = s_ref[...]
        dn = (((1,), (1,)), ((), ()))
        for j in range(_PACK_REP):
            x = x_ref[:, j * _PACK_BT:(j + 1) * _PACK_BT]
            eo = lax.dot_general(
                sel, x, dn, preferred_element_type=jnp.float32
            )
            o_ref[j * half:(j + 1) * half, 0:emb] = eo[0:half, :]
            o_ref[j * half:(j + 1) * half, emb:2 * emb] = eo[half:2 * half, :]

    return pl.pallas_call(
        body,
        grid=(grid,),
        in_specs=[
            pl.BlockSpec((emb, bt), lambda i: (0, i)),
            pl.BlockSpec((_PACK_BT, _PACK_BT), lambda i: (0, 0)),
        ],
        out_specs=pl.BlockSpec((bt // 2, 2 * emb), lambda i: (i, 0)),
        out_shape=jax.ShapeDtypeStruct((vocab // 2, 2 * emb), jnp.float32),
    )


def kernel(input_tokens, token_table, pos_table):
    b, s = input_tokens.shape
    vocab, emb = token_table.shape
    rows = b * s
    idx = input_tokens.astype(jnp.int32).reshape(rows // _IDXW, _IDXW)
    half = _PACK_BT // 2
    r_ids = lax.broadcasted_iota(jnp.int32, (half, _PACK_BT), 0)
    t_ids = lax.broadcasted_iota(jnp.int32, (half, _PACK_BT), 1)
    sel_e = (t_ids == 2 * r_ids).astype(jnp.float32)
    sel_o = (t_ids == 2 * r_ids + 1).astype(jnp.float32)
    sel = jnp.concatenate([sel_e, sel_o], axis=0)
    packed = _build_pack(vocab, emb)(token_table.T, sel)
    tbl_lin = packed.reshape(vocab, emb)
    out128 = _build_embed(rows, emb, s)(idx, tbl_lin, pos_table)
    return out128[:, :emb].reshape(b, s, emb)


# speedup vs baseline: 16.1470x; 1.1306x over previous
import functools

import jax
import jax.numpy as jnp
from jax import lax
from jax.experimental import pallas as pl
from jax.experimental.pallas import tpu as pltpu
from jax.experimental.pallas import tpu_sc as plsc

_LANES = 16
_IDXW = 100


@functools.lru_cache(maxsize=None)
def _build_embed(rows, emb, seq):
    info = plsc.get_sparse_core_info()
    nc, ns = info.num_cores, info.num_subcores
    nw = nc * ns
    assert rows % nw == 0
    rpw = rows // nw
    chunk = 4 * seq
    assert rpw % chunk == 0 and chunk % _IDXW == 0
    nch = rpw // chunk
    ng = chunk // _IDXW
    nvec = emb // _LANES
    assert emb % _LANES == 0
    nrep = chunk // seq

    mesh = plsc.VectorSubcoreMesh(core_axis_name="c", subcore_axis_name="s")

    @functools.partial(
        pl.kernel,
        mesh=mesh,
        compiler_params=pltpu.CompilerParams(use_tc_tiling_on_sc=False),
        out_type=jax.ShapeDtypeStruct((rows, 2 * emb), jnp.float32),
        scratch_types=[
            pltpu.VMEM((2 * ng, _IDXW), jnp.int32),
            pltpu.VMEM((2 * chunk, emb), jnp.float32),
            pltpu.VMEM((seq, emb), jnp.float32),
            pltpu.SemaphoreType.DMA,
            pltpu.SemaphoreType.DMA,
        ],
    )
    def k(idx_hbm, table_hbm, pos_hbm, out_hbm, idx_v, rows_v, pos_v, sem0, sem1):
        wid = lax.axis_index("s") * nc + lax.axis_index("c")
        base = wid * rpw
        sems = (sem0, sem1)
        pltpu.sync_copy(pos_hbm.at[pl.ds(0, seq)], pos_v)

        def start_chunk(c, p):
            irow = pl.multiple_of(base // _IDXW + c * ng, 8)
            pltpu.sync_copy(
                idx_hbm.at[pl.ds(irow, ng)], idx_v.at[pl.ds(p * ng, ng)]
            )
            return [
                pltpu.async_copy(
                    table_hbm.at[idx_v.at[p * ng + g]],
                    rows_v.at[pl.ds(p * chunk + g * _IDXW, _IDXW)],
                    sems[p],
                )
                for g in range(ng)
            ]

        pending = start_chunk(0, 0)
        for c in range(nch):
            p = c % 2
            for cp in pending:
                cp.wait()
            if c + 1 < nch:
                pending = start_chunk(c + 1, 1 - p)

            def add_body(s, carry):
                for e in range(nvec):
                    pv = pos_v[s, pl.ds(e * _LANES, _LANES)]
                    for q in range(nrep):
                        r = p * chunk + q * seq + s
                        rows_v[r, pl.ds(e * _LANES, _LANES)] = (
                            rows_v[r, pl.ds(e * _LANES, _LANES)] + pv
                        )
                return carry

            lax.fori_loop(0, seq, add_body, None)
            r0 = pl.multiple_of(base + c * chunk, 8)
            pltpu.sync_copy(
                rows_v.at[pl.ds(p * chunk, chunk)],
                out_hbm.at[pl.ds(r0, chunk), pl.ds(0, emb)],
            )

    return k


_PACK_BT = 128
_PACK_REP = 128


@functools.lru_cache(maxsize=None)
def _build_pack(vocab, emb):
    bt = _PACK_BT * _PACK_REP
    grid = (vocab + bt - 1) // bt
    half = _PACK_BT // 2

    def body(x_ref, s_ref, o_ref):
        sel = s_ref[...]
        dn = (((1,), (1,)), ((), ()))
        for j in range(_PACK_REP):
            x = x_ref[:, j * _PACK_BT:(j + 1) * _PACK_BT]
            eo = lax.dot_general(
                sel, x, dn, preferred_element_type=jnp.float32
            )
            o_ref[j * half:(j + 1) * half, 0:emb] = eo[0:half, :]
            o_ref[j * half:(j + 1) * half, emb:2 * emb] = eo[half:2 * half, :]

    return pl.pallas_call(
        body,
        grid=(grid,),
        in_specs=[
            pl.BlockSpec((emb, bt), lambda i: (0, i)),
            pl.BlockSpec((_PACK_BT, _PACK_BT), lambda i: (0, 0)),
        ],
        out_specs=pl.BlockSpec((bt // 2, 2 * emb), lambda i: (i, 0)),
        out_shape=jax.ShapeDtypeStruct((vocab // 2, 2 * emb), jnp.float32),
    )


def kernel(input_tokens, token_table, pos_table):
    b, s = input_tokens.shape
    vocab, emb = token_table.shape
    rows = b * s
    idx = input_tokens.astype(jnp.int32).reshape(rows // _IDXW, _IDXW)
    half = _PACK_BT // 2
    r_ids = lax.broadcasted_iota(jnp.int32, (half, _PACK_BT), 0)
    t_ids = lax.broadcasted_iota(jnp.int32, (half, _PACK_BT), 1)
    sel_e = (t_ids == 2 * r_ids).astype(jnp.float32)
    sel_o = (t_ids == 2 * r_ids + 1).astype(jnp.float32)
    sel = jnp.concatenate([sel_e, sel_o], axis=0)
    packed = _build_pack(vocab, emb)(token_table.T, sel)
    tbl_lin = packed.reshape(vocab, emb)
    out128 = _build_embed(rows, emb, s)(idx, tbl_lin, pos_table)
    return out128[:, :emb].reshape(b, s, emb)


# speedup vs baseline: 16.5156x; 1.0228x over previous
import functools

import jax
import jax.numpy as jnp
from jax import lax
from jax.experimental import pallas as pl
from jax.experimental.pallas import tpu as pltpu
from jax.experimental.pallas import tpu_sc as plsc

_LANES = 16
_IDXW = 100


@functools.lru_cache(maxsize=None)
def _build_embed(rows, emb, seq):
    info = plsc.get_sparse_core_info()
    nc, ns = info.num_cores, info.num_subcores
    nw = nc * ns
    assert rows % nw == 0
    rpw = rows // nw
    chunk = 4 * seq
    assert rpw % chunk == 0 and chunk % _IDXW == 0
    nch = rpw // chunk
    ng = chunk // _IDXW
    nvec = emb // _LANES
    assert emb % _LANES == 0
    nrep = chunk // seq

    mesh = plsc.VectorSubcoreMesh(core_axis_name="c", subcore_axis_name="s")

    @functools.partial(
        pl.kernel,
        mesh=mesh,
        compiler_params=pltpu.CompilerParams(use_tc_tiling_on_sc=False),
        out_type=jax.ShapeDtypeStruct((rows, 2 * emb), jnp.float32),
        scratch_types=[
            pltpu.VMEM((2 * ng, _IDXW), jnp.int32),
            pltpu.VMEM((2 * chunk, emb), jnp.float32),
            pltpu.VMEM((seq, emb), jnp.float32),
            pltpu.SemaphoreType.DMA,
            pltpu.SemaphoreType.DMA,
        ],
    )
    def k(idx_hbm, table_hbm, pos_hbm, out_hbm, idx_v, rows_v, pos_v, sem0, sem1):
        wid = lax.axis_index("s") * nc + lax.axis_index("c")
        base = wid * rpw
        sems = (sem0, sem1)
        pltpu.sync_copy(pos_hbm.at[pl.ds(0, seq)], pos_v)

        def start_chunk(c, p):
            irow = pl.multiple_of(base // _IDXW + c * ng, 8)
            pltpu.sync_copy(
                idx_hbm.at[pl.ds(irow, ng)], idx_v.at[pl.ds(p * ng, ng)]
            )
            return [
                pltpu.async_copy(
                    table_hbm.at[idx_v.at[p * ng + g]],
                    rows_v.at[pl.ds(p * chunk + g * _IDXW, _IDXW)],
                    sems[p],
                )
                for g in range(ng)
            ]

        pending = start_chunk(0, 0)
        for c in range(nch):
            p = c % 2
            for cp in pending:
                cp.wait()
            if c + 1 < nch:
                pending = start_chunk(c + 1, 1 - p)

            def add_body(s, carry):
                for e in range(nvec):
                    pv = pos_v[s, pl.ds(e * _LANES, _LANES)]
                    for q in range(nrep):
                        r = p * chunk + q * seq + s
                        rows_v[r, pl.ds(e * _LANES, _LANES)] = (
                            rows_v[r, pl.ds(e * _LANES, _LANES)] + pv
                        )
                return carry

            lax.fori_loop(0, seq, add_body, None)
            r0 = pl.multiple_of(base + c * chunk, 8)
            pltpu.sync_copy(
                rows_v.at[pl.ds(p * chunk, chunk)],
                out_hbm.at[pl.ds(r0, chunk), pl.ds(0, emb)],
            )

    return k


_PACK_BT = 128
_PACK_REP = 256


@functools.lru_cache(maxsize=None)
def _build_pack(vocab, emb):
    bt = _PACK_BT * _PACK_REP
    grid = (vocab + bt - 1) // bt
    half = _PACK_BT // 2

    def body(x_ref, s_ref, o_ref):
        sel = s_ref[...]
        dn = (((1,), (1,)), ((), ()))
        for j in range(_PACK_REP):
            x = x_ref[:, j * _PACK_BT:(j + 1) * _PACK_BT]
            eo = lax.dot_general(
                sel, x, dn, preferred_element_type=jnp.float32
            )
            o_ref[j * half:(j + 1) * half, 0:emb] = eo[0:half, :]
            o_ref[j * half:(j + 1) * half, emb:2 * emb] = eo[half:2 * half, :]

    return pl.pallas_call(
        body,
        grid=(grid,),
        in_specs=[
            pl.BlockSpec((emb, bt), lambda i: (0, i)),
            pl.BlockSpec((_PACK_BT, _PACK_BT), lambda i: (0, 0)),
        ],
        out_specs=pl.BlockSpec((bt // 2, 2 * emb), lambda i: (i, 0)),
        out_shape=jax.ShapeDtypeStruct((vocab // 2, 2 * emb), jnp.float32),
    )


def kernel(input_tokens, token_table, pos_table):
    b, s = input_tokens.shape
    vocab, emb = token_table.shape
    rows = b * s
    idx = input_tokens.astype(jnp.int32).reshape(rows // _IDXW, _IDXW)
    half = _PACK_BT // 2
    r_ids = lax.broadcasted_iota(jnp.int32, (half, _PACK_BT), 0)
    t_ids = lax.broadcasted_iota(jnp.int32, (half, _PACK_BT), 1)
    sel_e = (t_ids == 2 * r_ids).astype(jnp.float32)
    sel_o = (t_ids == 2 * r_ids + 1).astype(jnp.float32)
    sel = jnp.concatenate([sel_e, sel_o], axis=0)
    packed = _build_pack(vocab, emb)(token_table.T, sel)
    tbl_lin = packed.reshape(vocab, emb)
    out128 = _build_embed(rows, emb, s)(idx, tbl_lin, pos_table)
    return out128[:, :emb].reshape(b, s, emb)
